# Initial kernel scaffold; baseline (speedup 1.0000x reference)
import jax
import jax.numpy as jnp
from jax.experimental import pallas as pl


def kernel(x, edge_index, batch, W1, b1, W2, b2, Wfc, bfc):
    raise NotImplementedError("write your pallas kernel here")


# speedup vs baseline: 20.3667x; 20.3667x over previous
import functools

import jax
import jax.numpy as jnp
from jax import lax
from jax.experimental import pallas as pl
from jax.experimental.pallas import tpu as pltpu
from jax.experimental.pallas import tpu_sc as plsc

N = 100000
E = 1600000
HH = 16
NG = 512
NC, NS = 2, 16
NW = NC * NS
EROWS = 12800
EPAD = EROWS * 128
DUMP = N
ACC_ROWS = 100352
GROUP = 8
GROUPS = EROWS // (NW * GROUP)

_mesh = plsc.VectorSubcoreMesh(core_axis_name="c", subcore_axis_name="s")


@functools.partial(
    pl.kernel,
    mesh=_mesh,
    out_type=jax.ShapeDtypeStruct((NW, N // 16, 16), jnp.float32),
    compiler_params=pltpu.CompilerParams(needs_layout_passes=False, use_tc_tiling_on_sc=False),
    scratch_types=[
        pltpu.VMEM((GROUP, 128), jnp.int32),
        pltpu.VMEM((ACC_ROWS // 16, 16), jnp.float32),
    ],
)
def _sc_degree(dst_hbm, part_hbm, dst_v, hist_v):
    c = lax.axis_index("c")
    s = lax.axis_index("s")
    wid = s * NC + c
    zf = jnp.zeros((16,), jnp.float32)
    ones = jnp.ones((16,), jnp.float32)

    def zero(i, carry):
        hist_v[i, :] = zf
        return carry

    lax.fori_loop(0, ACC_ROWS // 16, zero, 0)

    def grp(j, carry):
        row0 = (j * NW + wid) * GROUP
        pltpu.sync_copy(dst_hbm.at[pl.ds(row0, GROUP)], dst_v)

        def upd(i, carry2):
            idx = dst_v[i // 8, pl.ds((i % 8) * 16, 16)]
            r = jax.lax.shift_right_logical(idx, 4)
            col = jax.lax.bitwise_and(idx, 15)
            plsc.addupdate_scatter(hist_v, [r, col], ones)
            return carry2

        lax.fori_loop(0, GROUP * 8, upd, 0)
        return carry

    lax.fori_loop(0, GROUPS, grp, 0)
    pltpu.sync_copy(hist_v.at[pl.ds(0, N // 16)], part_hbm.at[wid])


@functools.partial(
    pl.kernel,
    mesh=_mesh,
    out_type=(
        jax.ShapeDtypeStruct((N, HH), jnp.float32),
        jax.ShapeDtypeStruct((N, HH), jnp.float32),
    ),
    compiler_params=pltpu.CompilerParams(needs_layout_passes=False, use_tc_tiling_on_sc=False),
    scratch_types=[
        pltpu.VMEM((GROUP, 128), jnp.int32),
        pltpu.VMEM((GROUP, 128), jnp.int32),
        pltpu.VMEM((GROUP * 128, HH), jnp.float32),
        pltpu.VMEM_SHARED((ACC_ROWS, HH), jnp.float32),
        pltpu.SemaphoreType.DMA,
    ],
)
def _sc_aggregate(g0_hbm, g1_hbm, src_hbm, dst_hbm, out0_hbm, out1_hbm,
                  src_v, dst_v, rows_v, acc_sh, sem):
    c = lax.axis_index("c")
    s = lax.axis_index("s")
    wid = s * NC + c
    zf = jnp.zeros((16,), jnp.float32)

    def zbuf(i, carry):
        rows_v[i, :] = zf
        return carry

    lax.fori_loop(0, GROUP * 128, zbuf, 0)

    zbase = s * (ACC_ROWS // NS)

    def zacc(i, carry):
        pltpu.sync_copy(rows_v, acc_sh.at[pl.ds(zbase + i * (GROUP * 128), GROUP * 128)])
        return carry

    lax.fori_loop(0, 6, zacc, 0)
    pltpu.sync_copy(rows_v.at[pl.ds(0, 128)], acc_sh.at[pl.ds(zbase + 6144, 128)])
    plsc.subcore_barrier()

    def grp(j, carry):
        row0 = (j * NS + s) * GROUP
        pltpu.sync_copy(src_hbm.at[pl.ds(row0, GROUP)], src_v)
        pltpu.sync_copy(dst_hbm.at[pl.ds(row0, GROUP)], dst_v)

        @pl.when(c == 0)
        def _gather0():
            def fire(i, carry2):
                pltpu.async_copy(g0_hbm.at[src_v.at[i]],
                                 rows_v.at[pl.ds(i * 128, 128)], sem)
                return carry2

            lax.fori_loop(0, GROUP, fire, 0)

            def drain(i, carry2):
                pltpu.make_async_copy(g0_hbm.at[src_v.at[i]],
                                      rows_v.at[pl.ds(i * 128, 128)], sem).wait()
                return carry2

            lax.fori_loop(0, GROUP, drain, 0)

        @pl.when(c == 1)
        def _gather1():
            def fire(i, carry2):
                pltpu.async_copy(g1_hbm.at[src_v.at[i]],
                                 rows_v.at[pl.ds(i * 128, 128)], sem)
                return carry2

            lax.fori_loop(0, GROUP, fire, 0)

            def drain(i, carry2):
                pltpu.make_async_copy(g1_hbm.at[src_v.at[i]],
                                      rows_v.at[pl.ds(i * 128, 128)], sem).wait()
                return carry2

            lax.fori_loop(0, GROUP, drain, 0)

        def scat(i, carry2):
            pltpu.sync_copy(rows_v.at[pl.ds(i * 128, 128)],
                            acc_sh.at[dst_v.at[i]], add=True)
            return carry2

        lax.fori_loop(0, GROUP, scat, 0)
        return carry

    lax.fori_loop(0, EROWS // (NS * GROUP), grp, 0)
    plsc.subcore_barrier()

    wb = s * (N // NS)

    @pl.when(c == 0)
    def _wb0():
        pltpu.sync_copy(acc_sh.at[pl.ds(wb, N // NS)], out0_hbm.at[pl.ds(wb, N // NS)])

    @pl.when(c == 1)
    def _wb1():
        pltpu.sync_copy(acc_sh.at[pl.ds(wb, N // NS)], out1_hbm.at[pl.ds(wb, N // NS)])


_B1 = 5000


def _tc0_body(part_ref, dinv_ref):
    deg = jnp.sum(part_ref[...], axis=0, keepdims=True) + 1.0
    dinv_ref[...] = lax.rsqrt(deg)


def _tc0(part2):
    return pl.pallas_call(
        _tc0_body,
        grid=(1,),
        in_specs=[pl.BlockSpec((NW, N), lambda i: (0, 0))],
        out_specs=pl.BlockSpec((1, N), lambda i: (0, 0)),
        out_shape=jax.ShapeDtypeStruct((1, N), jnp.float32),
    )(part2)


def _tc1_body(x_ref, w1_ref, dinv_ref, g0_ref, g1_ref):
    dinv = dinv_ref[...]
    h = jnp.dot(x_ref[...], w1_ref[...], preferred_element_type=jnp.float32)
    g = h * dinv
    g0_ref[...] = g[:, :HH]
    g1_ref[...] = g[:, HH:]


def _tc1(x, W1, dinv):
    return pl.pallas_call(
        _tc1_body,
        grid=(N // _B1,),
        in_specs=[
            pl.BlockSpec((_B1, 5), lambda i: (i, 0)),
            pl.BlockSpec((5, 32), lambda i: (0, 0)),
            pl.BlockSpec((_B1, 1), lambda i: (i, 0)),
        ],
        out_specs=[
            pl.BlockSpec((_B1, HH), lambda i: (i, 0)),
            pl.BlockSpec((_B1, HH), lambda i: (i, 0)),
        ],
        out_shape=[
            jax.ShapeDtypeStruct((N, HH), jnp.float32),
            jax.ShapeDtypeStruct((N, HH), jnp.float32),
        ],
    )(x, W1, dinv)


def _tc2_body(a0_ref, a1_ref, g0_ref, g1_ref, dinv_ref, b1_ref, w2_ref,
              o0_ref, o1_ref):
    dinv = dinv_ref[...]
    a = jnp.concatenate([a0_ref[...] + g0_ref[...],
                         a1_ref[...] + g1_ref[...]], axis=1)
    h = jnp.maximum(a * dinv + b1_ref[...], 0.0)
    g2 = jnp.dot(h, w2_ref[...], preferred_element_type=jnp.float32) * dinv
    o0_ref[...] = g2[:, :HH]
    o1_ref[...] = g2[:, HH:]


def _tc2(agg0, agg1, g0, g1, dinv, b1, W2):
    return pl.pallas_call(
        _tc2_body,
        grid=(N // _B1,),
        in_specs=[
            pl.BlockSpec((_B1, HH), lambda i: (i, 0)),
            pl.BlockSpec((_B1, HH), lambda i: (i, 0)),
            pl.BlockSpec((_B1, HH), lambda i: (i, 0)),
            pl.BlockSpec((_B1, HH), lambda i: (i, 0)),
            pl.BlockSpec((_B1, 1), lambda i: (i, 0)),
            pl.BlockSpec((1, 32), lambda i: (0, 0)),
            pl.BlockSpec((32, 32), lambda i: (0, 0)),
        ],
        out_specs=[
            pl.BlockSpec((_B1, HH), lambda i: (i, 0)),
            pl.BlockSpec((_B1, HH), lambda i: (i, 0)),
        ],
        out_shape=[
            jax.ShapeDtypeStruct((N, HH), jnp.float32),
            jax.ShapeDtypeStruct((N, HH), jnp.float32),
        ],
    )(agg0, agg1, g0, g1, dinv, b1, W2)


_B3 = 2000


def _tc3_body(a0_ref, a1_ref, g0_ref, g1_ref, dinv_ref, b2_ref, batch_ref,
              wfc_ref, bfc_ref, out_ref, sums, cnts):
    i = pl.program_id(0)

    @pl.when(i == 0)
    def _init():
        sums[...] = jnp.zeros_like(sums)
        cnts[...] = jnp.zeros_like(cnts)

    a = jnp.concatenate([a0_ref[...] + g0_ref[...],
                         a1_ref[...] + g1_ref[...]], axis=1)
    h = a * dinv_ref[...] + b2_ref[...]
    onehot = (batch_ref[...] ==
              lax.broadcasted_iota(jnp.int32, (_B3, NG), 1)).astype(jnp.float32)
    sums[...] += lax.dot_general(onehot, h, (((0,), (0,)), ((), ())),
                                 preferred_element_type=jnp.float32)
    cnts[...] += lax.dot_general(onehot, jnp.ones((_B3, 1), jnp.float32),
                                 (((0,), (0,)), ((), ())),
                                 preferred_element_type=jnp.float32)
    pooled = sums[...] / jnp.maximum(cnts[...], 1.0)
    out_ref[...] = jnp.dot(pooled, wfc_ref[...],
                           preferred_element_type=jnp.float32) + bfc_ref[...]


def _tc3(agg0, agg1, g0, g1, dinv, b2, batch2, Wfc, bfc):
    return pl.pallas_call(
        _tc3_body,
        grid=(N // _B3,),
        in_specs=[
            pl.BlockSpec((_B3, HH), lambda i: (i, 0)),
            pl.BlockSpec((_B3, HH), lambda i: (i, 0)),
            pl.BlockSpec((_B3, HH), lambda i: (i, 0)),
            pl.BlockSpec((_B3, HH), lambda i: (i, 0)),
            pl.BlockSpec((_B3, 1), lambda i: (i, 0)),
            pl.BlockSpec((1, 32), lambda i: (0, 0)),
            pl.BlockSpec((_B3, 1), lambda i: (i, 0)),
            pl.BlockSpec((32, 2), lambda i: (0, 0)),
            pl.BlockSpec((1, 2), lambda i: (0, 0)),
        ],
        out_specs=pl.BlockSpec((NG, 2), lambda i: (0, 0)),
        out_shape=jax.ShapeDtypeStruct((NG, 2), jnp.float32),
        scratch_shapes=[
            pltpu.VMEM((NG, 32), jnp.float32),
            pltpu.VMEM((NG, 1), jnp.float32),
        ],
    )(agg0, agg1, g0, g1, dinv, b2, batch2, Wfc, bfc)


def kernel(x, edge_index, batch, W1, b1, W2, b2, Wfc, bfc):
    src = edge_index[0].astype(jnp.int32)
    dst = edge_index[1].astype(jnp.int32)
    pad = EPAD - E
    src_p = jnp.concatenate([src, jnp.zeros((pad,), jnp.int32)]).reshape(EROWS, 128)
    dst_p = jnp.concatenate([dst, jnp.full((pad,), DUMP, jnp.int32)]).reshape(EROWS, 128)
    batch2 = batch.astype(jnp.int32).reshape(N, 1)
    b1r = b1.reshape(1, 32)
    b2r = b2.reshape(1, 32)
    bfcr = bfc.reshape(1, 2)

    part = _sc_degree(dst_p)
    dinv = _tc0(part.reshape(NW, N)).reshape(N, 1)
    g0, g1 = _tc1(x, W1, dinv)
    agg0, agg1 = _sc_aggregate(g0, g1, src_p, dst_p)
    g2_0, g2_1 = _tc2(agg0, agg1, g0, g1, dinv, b1r, W2)
    agg0b, agg1b = _sc_aggregate(g2_0, g2_1, src_p, dst_p)
    return _tc3(agg0b, agg1b, g2_0, g2_1, dinv, b2r, batch2, Wfc, bfcr)


# speedup vs baseline: 23.6470x; 1.1611x over previous
import functools

import jax
import jax.numpy as jnp
from jax import lax
from jax.experimental import pallas as pl
from jax.experimental.pallas import tpu as pltpu
from jax.experimental.pallas import tpu_sc as plsc

N = 100000
E = 1600000
HH = 16
NG = 512
NC, NS = 2, 16
NW = NC * NS
EROWS = 12800
EPAD = EROWS * 128
DUMP = N
ACC_ROWS = 100352
GROUP = 8
GROUPS = EROWS // (NW * GROUP)
RG = 5
NGRP = EROWS // (NS * RG)

_mesh = plsc.VectorSubcoreMesh(core_axis_name="c", subcore_axis_name="s")


@functools.partial(
    pl.kernel,
    mesh=_mesh,
    out_type=jax.ShapeDtypeStruct((NW, N // 16, 16), jnp.float32),
    compiler_params=pltpu.CompilerParams(needs_layout_passes=False, use_tc_tiling_on_sc=False),
    scratch_types=[
        pltpu.VMEM((GROUP, 128), jnp.int32),
        pltpu.VMEM((ACC_ROWS // 16, 16), jnp.float32),
    ],
)
def _sc_degree(dst_hbm, part_hbm, dst_v, hist_v):
    c = lax.axis_index("c")
    s = lax.axis_index("s")
    wid = s * NC + c
    zf = jnp.zeros((16,), jnp.float32)
    ones = jnp.ones((16,), jnp.float32)

    def zero(i, carry):
        hist_v[i, :] = zf
        return carry

    lax.fori_loop(0, ACC_ROWS // 16, zero, 0)

    def grp(j, carry):
        row0 = (j * NW + wid) * GROUP
        pltpu.sync_copy(dst_hbm.at[pl.ds(row0, GROUP)], dst_v)

        def upd(i, carry2):
            idx = dst_v[i // 8, pl.ds((i % 8) * 16, 16)]
            r = jax.lax.shift_right_logical(idx, 4)
            col = jax.lax.bitwise_and(idx, 15)
            plsc.addupdate_scatter(hist_v, [r, col], ones)
            return carry2

        lax.fori_loop(0, GROUP * 8, upd, 0)
        return carry

    lax.fori_loop(0, GROUPS, grp, 0)
    pltpu.sync_copy(hist_v.at[pl.ds(0, N // 16)], part_hbm.at[wid])


@functools.partial(
    pl.kernel,
    mesh=_mesh,
    out_type=(
        jax.ShapeDtypeStruct((N, HH), jnp.float32),
        jax.ShapeDtypeStruct((N, HH), jnp.float32),
    ),
    compiler_params=pltpu.CompilerParams(needs_layout_passes=False, use_tc_tiling_on_sc=False),
    scratch_types=[
        pltpu.VMEM((RG, 128), jnp.int32),
        pltpu.VMEM((RG, 128), jnp.int32),
        pltpu.VMEM((RG, 128), jnp.int32),
        pltpu.VMEM((RG, 128), jnp.int32),
        pltpu.VMEM((RG * 128, HH), jnp.float32),
        pltpu.VMEM((RG * 128, HH), jnp.float32),
        pltpu.VMEM_SHARED((ACC_ROWS, HH), jnp.float32),
        pltpu.SemaphoreType.DMA,
        pltpu.SemaphoreType.DMA,
        pltpu.SemaphoreType.DMA,
    ],
)
def _sc_aggregate(g0_hbm, g1_hbm, src_hbm, dst_hbm, out0_hbm, out1_hbm,
                  src_a, dst_a, src_b, dst_b, rows_a, rows_b, acc_sh,
                  sem_ga, sem_gb, sem_s):
    c = lax.axis_index("c")
    s = lax.axis_index("s")
    zf = jnp.zeros((16,), jnp.float32)

    def zbuf(i, carry):
        rows_a[i, :] = zf
        return carry

    lax.fori_loop(0, RG * 128, zbuf, 0)

    zbase = s * (ACC_ROWS // NS)

    def zacc(i, carry):
        pltpu.sync_copy(rows_a, acc_sh.at[pl.ds(zbase + i * (RG * 128), RG * 128)])
        return carry

    lax.fori_loop(0, 9, zacc, 0)
    pltpu.sync_copy(rows_a.at[pl.ds(0, 512)], acc_sh.at[pl.ds(zbase + 9 * RG * 128, 512)])
    plsc.subcore_barrier()

    def load_idx(g, src_v, dst_v):
        row0 = (g * NS + s) * RG
        pltpu.sync_copy(src_hbm.at[pl.ds(row0, RG)], src_v)
        pltpu.sync_copy(dst_hbm.at[pl.ds(row0, RG)], dst_v)

    def fire(tbl, src_v, rows_v, sem):
        def f(i, carry):
            pltpu.async_copy(tbl.at[src_v.at[i]],
                             rows_v.at[pl.ds(i * 128, 128)], sem)
            return carry

        lax.fori_loop(0, RG, f, 0)

    def gather(src_v, rows_v, sem):
        @pl.when(c == 0)
        def _g0():
            fire(g0_hbm, src_v, rows_v, sem)

        @pl.when(c == 1)
        def _g1():
            fire(g1_hbm, src_v, rows_v, sem)

    def drain(src_v, rows_v, sem):
        def f(i, carry):
            pltpu.make_async_copy(g0_hbm.at[src_v.at[i]],
                                  rows_v.at[pl.ds(i * 128, 128)], sem).wait()
            return carry

        lax.fori_loop(0, RG, f, 0)

    def scatter(dst_v, rows_v):
        def f(i, carry):
            pltpu.async_copy(rows_v.at[pl.ds(i * 128, 128)],
                             acc_sh.at[dst_v.at[i]], sem_s, add=True)
            return carry

        lax.fori_loop(0, RG, f, 0)

        def d(i, carry):
            pltpu.make_async_copy(rows_v.at[pl.ds(i * 128, 128)],
                                  acc_sh.at[dst_v.at[i]], sem_s).wait()
            return carry

        lax.fori_loop(0, RG, d, 0)

    load_idx(0, src_a, dst_a)
    gather(src_a, rows_a, sem_ga)

    def body(k, carry):
        g1 = 2 * k + 1
        load_idx(g1, src_b, dst_b)
        gather(src_b, rows_b, sem_gb)
        drain(src_a, rows_a, sem_ga)
        scatter(dst_a, rows_a)

        @pl.when(k < NGRP // 2 - 1)
        def _next():
            load_idx(2 * k + 2, src_a, dst_a)
            gather(src_a, rows_a, sem_ga)

        drain(src_b, rows_b, sem_gb)
        scatter(dst_b, rows_b)
        return carry

    lax.fori_loop(0, NGRP // 2, body, 0)
    plsc.subcore_barrier()

    wb = s * (N // NS)

    @pl.when(c == 0)
    def _wb0():
        pltpu.sync_copy(acc_sh.at[pl.ds(wb, N // NS)], out0_hbm.at[pl.ds(wb, N // NS)])

    @pl.when(c == 1)
    def _wb1():
        pltpu.sync_copy(acc_sh.at[pl.ds(wb, N // NS)], out1_hbm.at[pl.ds(wb, N // NS)])


_B1 = 5000


def _tc0_body(part_ref, dinv_ref):
    deg = jnp.sum(part_ref[...], axis=0, keepdims=True) + 1.0
    dinv_ref[...] = lax.rsqrt(deg)


def _tc0(part2):
    return pl.pallas_call(
        _tc0_body,
        grid=(1,),
        in_specs=[pl.BlockSpec((NW, N), lambda i: (0, 0))],
        out_specs=pl.BlockSpec((1, N), lambda i: (0, 0)),
        out_shape=jax.ShapeDtypeStruct((1, N), jnp.float32),
    )(part2)


def _tc1_body(x_ref, w1_ref, dinv_ref, g0_ref, g1_ref):
    dinv = dinv_ref[...]
    h = jnp.dot(x_ref[...], w1_ref[...], preferred_element_type=jnp.float32)
    g = h * dinv
    g0_ref[...] = g[:, :HH]
    g1_ref[...] = g[:, HH:]


def _tc1(x, W1, dinv):
    return pl.pallas_call(
        _tc1_body,
        grid=(N // _B1,),
        in_specs=[
            pl.BlockSpec((_B1, 5), lambda i: (i, 0)),
            pl.BlockSpec((5, 32), lambda i: (0, 0)),
            pl.BlockSpec((_B1, 1), lambda i: (i, 0)),
        ],
        out_specs=[
            pl.BlockSpec((_B1, HH), lambda i: (i, 0)),
            pl.BlockSpec((_B1, HH), lambda i: (i, 0)),
        ],
        out_shape=[
            jax.ShapeDtypeStruct((N, HH), jnp.float32),
            jax.ShapeDtypeStruct((N, HH), jnp.float32),
        ],
    )(x, W1, dinv)


def _tc2_body(a0_ref, a1_ref, g0_ref, g1_ref, dinv_ref, b1_ref, w2_ref,
              o0_ref, o1_ref):
    dinv = dinv_ref[...]
    a = jnp.concatenate([a0_ref[...] + g0_ref[...],
                         a1_ref[...] + g1_ref[...]], axis=1)
    h = jnp.maximum(a * dinv + b1_ref[...], 0.0)
    g2 = jnp.dot(h, w2_ref[...], preferred_element_type=jnp.float32) * dinv
    o0_ref[...] = g2[:, :HH]
    o1_ref[...] = g2[:, HH:]


def _tc2(agg0, agg1, g0, g1, dinv, b1, W2):
    return pl.pallas_call(
        _tc2_body,
        grid=(N // _B1,),
        in_specs=[
            pl.BlockSpec((_B1, HH), lambda i: (i, 0)),
            pl.BlockSpec((_B1, HH), lambda i: (i, 0)),
            pl.BlockSpec((_B1, HH), lambda i: (i, 0)),
            pl.BlockSpec((_B1, HH), lambda i: (i, 0)),
            pl.BlockSpec((_B1, 1), lambda i: (i, 0)),
            pl.BlockSpec((1, 32), lambda i: (0, 0)),
            pl.BlockSpec((32, 32), lambda i: (0, 0)),
        ],
        out_specs=[
            pl.BlockSpec((_B1, HH), lambda i: (i, 0)),
            pl.BlockSpec((_B1, HH), lambda i: (i, 0)),
        ],
        out_shape=[
            jax.ShapeDtypeStruct((N, HH), jnp.float32),
            jax.ShapeDtypeStruct((N, HH), jnp.float32),
        ],
    )(agg0, agg1, g0, g1, dinv, b1, W2)


_B3 = 2000


def _tc3_body(a0_ref, a1_ref, g0_ref, g1_ref, dinv_ref, b2_ref, batch_ref,
              wfc_ref, bfc_ref, out_ref, sums, cnts):
    i = pl.program_id(0)

    @pl.when(i == 0)
    def _init():
        sums[...] = jnp.zeros_like(sums)
        cnts[...] = jnp.zeros_like(cnts)

    a = jnp.concatenate([a0_ref[...] + g0_ref[...],
                         a1_ref[...] + g1_ref[...]], axis=1)
    h = a * dinv_ref[...] + b2_ref[...]
    onehot = (batch_ref[...] ==
              lax.broadcasted_iota(jnp.int32, (_B3, NG), 1)).astype(jnp.float32)
    sums[...] += lax.dot_general(onehot, h, (((0,), (0,)), ((), ())),
                                 preferred_element_type=jnp.float32)
    cnts[...] += lax.dot_general(onehot, jnp.ones((_B3, 1), jnp.float32),
                                 (((0,), (0,)), ((), ())),
                                 preferred_element_type=jnp.float32)
    pooled = sums[...] / jnp.maximum(cnts[...], 1.0)
    out_ref[...] = jnp.dot(pooled, wfc_ref[...],
                           preferred_element_type=jnp.float32) + bfc_ref[...]


def _tc3(agg0, agg1, g0, g1, dinv, b2, batch2, Wfc, bfc):
    return pl.pallas_call(
        _tc3_body,
        grid=(N // _B3,),
        in_specs=[
            pl.BlockSpec((_B3, HH), lambda i: (i, 0)),
            pl.BlockSpec((_B3, HH), lambda i: (i, 0)),
            pl.BlockSpec((_B3, HH), lambda i: (i, 0)),
            pl.BlockSpec((_B3, HH), lambda i: (i, 0)),
            pl.BlockSpec((_B3, 1), lambda i: (i, 0)),
            pl.BlockSpec((1, 32), lambda i: (0, 0)),
            pl.BlockSpec((_B3, 1), lambda i: (i, 0)),
            pl.BlockSpec((32, 2), lambda i: (0, 0)),
            pl.BlockSpec((1, 2), lambda i: (0, 0)),
        ],
        out_specs=pl.BlockSpec((NG, 2), lambda i: (0, 0)),
        out_shape=jax.ShapeDtypeStruct((NG, 2), jnp.float32),
        scratch_shapes=[
            pltpu.VMEM((NG, 32), jnp.float32),
            pltpu.VMEM((NG, 1), jnp.float32),
        ],
    )(agg0, agg1, g0, g1, dinv, b2, batch2, Wfc, bfc)


def kernel(x, edge_index, batch, W1, b1, W2, b2, Wfc, bfc):
    src = edge_index[0].astype(jnp.int32)
    dst = edge_index[1].astype(jnp.int32)
    pad = EPAD - E
    src_p = jnp.concatenate([src, jnp.zeros((pad,), jnp.int32)]).reshape(EROWS, 128)
    dst_p = jnp.concatenate([dst, jnp.full((pad,), DUMP, jnp.int32)]).reshape(EROWS, 128)
    batch2 = batch.astype(jnp.int32).reshape(N, 1)
    b1r = b1.reshape(1, 32)
    b2r = b2.reshape(1, 32)
    bfcr = bfc.reshape(1, 2)

    part = _sc_degree(dst_p)
    dinv = _tc0(part.reshape(NW, N)).reshape(N, 1)
    g0, g1 = _tc1(x, W1, dinv)
    agg0, agg1 = _sc_aggregate(g0, g1, src_p, dst_p)
    g2_0, g2_1 = _tc2(agg0, agg1, g0, g1, dinv, b1r, W2)
    agg0b, agg1b = _sc_aggregate(g2_0, g2_1, src_p, dst_p)
    return _tc3(agg0b, agg1b, g2_0, g2_1, dinv, b2r, batch2, Wfc, bfcr)
